# baseline (speedup 1.0000x reference)
import jax
import jax.numpy as jnp
from jax import lax
from jax.experimental import pallas as pl
from jax.experimental.pallas import tpu as pltpu
from jax.experimental.pallas import tpu_sc as plsc

B, N, E = 4, 10000, 160000
C = 256
K = 5
EPS = 1e-5
POOL = 4
LAMBDA_MAX = 2.0

NC, NS = 2, 16
Q = B * 2
QN = Q * N

EB = 128
PAD_E = 163840
TPB_A = PAD_E // NS // EB
TPB_B = PAD_E // (NC * NS) // EB
NPT = N // NS
NPC = 125

_SCALE = 2.0 / LAMBDA_MAX

_MESH = plsc.VectorSubcoreMesh(
    core_axis_name="c", subcore_axis_name="s", num_cores=NC, num_subcores=NS)


def _iota16():
    return lax.iota(jnp.int32, 16)


def _rsqrt_sc(x):
    xi = plsc.bitcast(x, jnp.int32)
    y = plsc.bitcast(jnp.int32(0x5F3759DF) - (xi >> 1), jnp.float32)
    for _ in range(3):
        y = y * (1.5 - 0.5 * x * y * y)
    return y


NP = 10240
MY = NP // NS
EBIG = 1024
OFF_CH = PAD_E // (NC * NS) // EBIG


def _dega_body(src_hbm, dst_hbm, w_hbm, deg_hbm,
               sh_deg, bsrc, bdst, bw, bwm, rows):
    c = lax.axis_index("c")
    s = lax.axis_index("s")

    def zr(i, _):
        for u in range(8):
            rows[i, pl.ds(u * 16, 16)] = jnp.zeros((16,), jnp.float32)
        return _
    lax.fori_loop(0, EB, zr, 0)
    for p in range(MY // EB):
        pltpu.sync_copy(
            rows, sh_deg.at[pl.ds(pl.multiple_of(s * MY + p * EB, 8), EB)])
    plsc.subcore_barrier()

    def deg_batch(i, _):
        base = pl.multiple_of((c * NS + s) * (TPB_B * EB) + i * EB, 8)
        pltpu.sync_copy(src_hbm.at[pl.ds(base, EB)], bsrc)
        pltpu.sync_copy(dst_hbm.at[pl.ds(base, EB)], bdst)
        pltpu.sync_copy(w_hbm.at[pl.ds(base, EB)], bw)
        for g in range(EB // 16):
            sl = pl.ds(g * 16, 16)
            bwm[sl] = jnp.where(bsrc[sl] == bdst[sl], 0.0, bw[sl])

        def fill(e16, _):
            for t in range(16):
                e = e16 * 16 + t
                wv = plsc.load_gather(bwm, [jnp.broadcast_to(e, (16,))])
                for u in range(8):
                    rows[e, pl.ds(u * 16, 16)] = wv
            return _
        lax.fori_loop(0, EB // 16, fill, 0)
        pltpu.sync_copy(rows, sh_deg.at[bsrc], add=True)
        return _
    lax.fori_loop(0, TPB_B, deg_batch, 0)
    plsc.subcore_barrier()

    my = pl.multiple_of(s * MY, 8)
    pltpu.sync_copy(sh_deg.at[pl.ds(my, MY)],
                    deg_hbm.at[pl.ds(pl.multiple_of(c * NP + my, 8), MY)])


def _sc_dega(src, dst, w):
    return pl.kernel(
        _dega_body,
        out_type=jax.ShapeDtypeStruct((NC * NP, 128), jnp.float32),
        mesh=_MESH,
        compiler_params=pltpu.CompilerParams(needs_layout_passes=False),
        scratch_types=[
            pltpu.VMEM_SHARED((NP, 128), jnp.float32),
            pltpu.VMEM((EB,), jnp.int32),
            pltpu.VMEM((EB,), jnp.int32),
            pltpu.VMEM((EB,), jnp.float32),
            pltpu.VMEM((EB,), jnp.float32),
            pltpu.VMEM((EB, 128), jnp.float32),
        ],
    )(src, dst, w)


def _dinv_body(d_ref, o_ref):
    d = d_ref[0:NP, 0:1] + d_ref[NP:2 * NP, 0:1]
    r = lax.rsqrt(d)
    o_ref[...] = jnp.where(d > 0.0, r, 0.0)


def _tc_dinv(degrows):
    return pl.pallas_call(
        _dinv_body,
        in_specs=[pl.BlockSpec((NC * NP, 128), lambda: (0, 0))],
        out_specs=pl.BlockSpec((NP, 1), lambda: (0, 0)),
        out_shape=jax.ShapeDtypeStruct((NP, 1), jnp.float32),
    )(degrows)


def _off_body(src_hbm, dst_hbm, w_hbm, dinv_hbm, off_hbm,
              bsrc, bdst, bw, dinv_loc, boff):
    c = lax.axis_index("c")
    s = lax.axis_index("s")
    pltpu.sync_copy(dinv_hbm, dinv_loc)
    wid = c * NS + s
    def off_batch(i, _):
        ebase = pl.multiple_of(wid * (OFF_CH * EBIG) + i * EBIG, 8)
        pltpu.sync_copy(src_hbm.at[pl.ds(ebase, EBIG)], bsrc)
        pltpu.sync_copy(dst_hbm.at[pl.ds(ebase, EBIG)], bdst)
        pltpu.sync_copy(w_hbm.at[pl.ds(ebase, EBIG)], bw)
        def grp(g, _):
            sl = pl.ds(g * 16, 16)
            sv = bsrc[sl]
            dv = bdst[sl]
            wm = jnp.where(sv == dv, 0.0, bw[sl])
            g1 = plsc.load_gather(dinv_loc, [sv])
            g2 = plsc.load_gather(dinv_loc, [dv])
            boff[sl] = (-_SCALE) * g1 * wm * g2
            return _
        lax.fori_loop(0, EBIG // 16, grp, 0)
        pltpu.sync_copy(boff, off_hbm.at[pl.ds(ebase, EBIG)])
        return _
    lax.fori_loop(0, OFF_CH, off_batch, 0)


def _sc_off(src, dst, w, dinv):
    return pl.kernel(
        _off_body,
        out_type=jax.ShapeDtypeStruct((PAD_E,), jnp.float32),
        mesh=_MESH,
        compiler_params=pltpu.CompilerParams(needs_layout_passes=False),
        scratch_types=[
            pltpu.VMEM((EBIG,), jnp.int32),
            pltpu.VMEM((EBIG,), jnp.int32),
            pltpu.VMEM((EBIG,), jnp.float32),
            pltpu.VMEM((NP,), jnp.float32),
            pltpu.VMEM((EBIG,), jnp.float32),
        ],
    )(src, dst, w, dinv)


NSLOT = 4
EPT = PAD_E // NS
PEB = 80
EPKW = 384
NBT = EPT // PEB


def _prop_body(y_hbm, epk_hbm, s_hbm,
               sh_acc, gidx4, bdst4, bpk4, rows4,
               gs0, gs1, gs2, gs3, ss0, ss1, ss2, ss3,
               ps0, ps1, ps2, ps3):
    c = lax.axis_index("c")
    s = lax.axis_index("s")
    gsems = (gs0, gs1, gs2, gs3)
    ssems = (ss0, ss1, ss2, ss3)
    psems = (ps0, ps1, ps2, ps3)

    my0 = pl.multiple_of(s * 624, 8)
    gb0 = s * NBT

    def pstart(ib, k):
        pltpu.async_copy(epk_hbm.at[gb0 + ib], bpk4.at[k], psems[k])

    def pwait(k):
        pltpu.make_async_copy(epk_hbm.at[gb0], bpk4.at[k], psems[k]).wait()


    def gstart(k):
        pltpu.async_copy(y_hbm.at[gidx4.at[k]], rows4.at[k], gsems[k])

    def gwait(k):
        pltpu.make_async_copy(y_hbm.at[gidx4.at[k]], rows4.at[k],
                              gsems[k]).wait()

    def sstart(k):
        pltpu.async_copy(rows4.at[k], sh_acc.at[bdst4.at[k]], ssems[k],
                         add=True)

    def swait(k):
        pltpu.make_async_copy(rows4.at[k], sh_acc.at[bdst4.at[k]],
                              ssems[k]).wait()

    def scale(k):
        k16 = jnp.full((16,), k, jnp.int32)

        def scale_e16(e16, _):
            for t in range(16):
                e = e16 * 16 + t
                ov = plsc.bitcast(
                    plsc.load_gather(
                        bpk4, [k16, jnp.broadcast_to(2 * PEB + e, (16,))]),
                    jnp.float32)
                for u in range(8):
                    cs = pl.ds(u * 16, 16)
                    rows4[k, e, cs] = rows4[k, e, cs] * ov
            return _
        lax.fori_loop(0, PEB // 16, scale_e16, 0)

    def chunk(j, carry):
        q = c * (Q // NC) + j
        qb = pl.multiple_of(q * N, 8)
        qn = q * N

        def zz(r, _):
            for u in range(8):
                rows4[0, r, pl.ds(u * 16, 16)] = jnp.zeros((16,), jnp.float32)
            return _
        lax.fori_loop(0, PEB, zz, 0)

        @pl.when(s < NS - 1)
        def _():
            for p in range(7):
                pltpu.async_copy(rows4.at[0],
                                 sh_acc.at[pl.ds(my0 + p * PEB, PEB)], gs0)
            pltpu.async_copy(rows4.at[0, pl.ds(0, 64)],
                             sh_acc.at[pl.ds(my0 + 560, 64)], gs0)
            for p in range(7):
                pltpu.make_async_copy(
                    rows4.at[0],
                    sh_acc.at[pl.ds(my0 + p * PEB, PEB)], gs0).wait()
            pltpu.make_async_copy(rows4.at[0, pl.ds(0, 64)],
                                  sh_acc.at[pl.ds(my0 + 560, 64)], gs0).wait()

        @pl.when(s == NS - 1)
        def _():
            for p in range(8):
                pltpu.async_copy(rows4.at[0],
                                 sh_acc.at[pl.ds(9360 + p * PEB, PEB)], gs0)
            for p in range(8):
                pltpu.make_async_copy(
                    rows4.at[0],
                    sh_acc.at[pl.ds(9360 + p * PEB, PEB)], gs0).wait()
        plsc.subcore_barrier()

        def buildq(ib, k):
            for g in range(PEB // 16):
                sl = pl.ds(g * 16, 16)
                gidx4[k, sl] = bpk4[k, pl.ds(g * 16, 16)] + qn
                bdst4[k, sl] = bpk4[k, pl.ds(PEB + g * 16, 16)]

        for k in range(NSLOT):
            pstart(jnp.int32(k), k)
        for k in range(2):
            pwait(k)
            buildq(jnp.int32(k), k)
            gstart(k)

        def quad(i4, carry2):
            for u in range(NSLOT):
                i = i4 * NSLOT + u
                kn = (u + 2) % NSLOT

                @pl.when(i >= 2)
                def _():
                    swait(kn)

                @pl.when(i + 2 < NBT)
                def _():
                    pwait(kn)
                    buildq(i + 2, kn)
                    gstart(kn)
                gwait(u)
                scale(u)
                sstart(u)

                @pl.when(i + NSLOT < NBT)
                def _():
                    pstart(i + NSLOT, u)
            return carry2
        lax.fori_loop(0, NBT // NSLOT, quad, 0)
        for k in (2, 3):
            swait(k)
        plsc.subcore_barrier()

        @pl.when(s < NS - 1)
        def _():
            pltpu.sync_copy(sh_acc.at[pl.ds(my0, 624)],
                            s_hbm.at[pl.ds(pl.multiple_of(qb + my0, 8), 624)])

        @pl.when(s == NS - 1)
        def _():
            pltpu.sync_copy(sh_acc.at[pl.ds(9360, 640)],
                            s_hbm.at[pl.ds(pl.multiple_of(qb + 9360, 8), 640)])
        plsc.subcore_barrier()
        return carry
    lax.fori_loop(0, Q // NC, chunk, 0)


def _sc_propagate(y2d, epk):
    return pl.kernel(
        _prop_body,
        out_type=jax.ShapeDtypeStruct((QN, 128), jnp.float32),
        mesh=_MESH,
        compiler_params=pltpu.CompilerParams(needs_layout_passes=False),
        scratch_types=[
            pltpu.VMEM_SHARED((N, 128), jnp.float32),
            pltpu.VMEM((NSLOT, PEB), jnp.int32),
            pltpu.VMEM((NSLOT, PEB), jnp.int32),
            pltpu.VMEM((NSLOT, EPKW), jnp.int32),
            pltpu.VMEM((NSLOT, PEB, 128), jnp.float32),
            pltpu.SemaphoreType.DMA, pltpu.SemaphoreType.DMA,
            pltpu.SemaphoreType.DMA, pltpu.SemaphoreType.DMA,
            pltpu.SemaphoreType.DMA, pltpu.SemaphoreType.DMA,
            pltpu.SemaphoreType.DMA, pltpu.SemaphoreType.DMA,
            pltpu.SemaphoreType.DMA, pltpu.SemaphoreType.DMA,
            pltpu.SemaphoreType.DMA, pltpu.SemaphoreType.DMA,
        ],
    )(y2d, epk)


BN_ROWS = 2000
NT = N // BN_ROWS


def _transpose_body(x_ref, o_ref):
    o_ref[...] = x_ref[...]


def _tc_transpose(x):
    return pl.pallas_call(
        _transpose_body,
        grid=(Q, NT),
        in_specs=[pl.BlockSpec((1, BN_ROWS, 128),
                               lambda q, n: (q // 2, n, q % 2))],
        out_specs=pl.BlockSpec((1, BN_ROWS, 128), lambda q, n: (q, n, 0)),
        out_shape=jax.ShapeDtypeStruct((Q, N, 128), jnp.float32),
    )(x)


def _combine_body(s_ref, y0_ref, o_ref):
    o_ref[...] = 2.0 * s_ref[...] - y0_ref[...]


def _tc_combine(s_new, y0):
    spec = pl.BlockSpec((1, BN_ROWS, 128), lambda q, n: (q, n, 0))
    return pl.pallas_call(
        _combine_body,
        grid=(Q, NT),
        in_specs=[spec, spec],
        out_specs=spec,
        out_shape=jax.ShapeDtypeStruct((Q, N, 128), jnp.float32),
    )(s_new, y0)


def _matmul_body(t0, t1, t2, t3, t4, w_ref, o_ref, st_ref):
    b = pl.program_id(0)
    n = pl.program_id(1)
    h = pl.program_id(2)

    @pl.when(jnp.logical_and(jnp.logical_and(b == 0, n == 0), h == 0))
    def _():
        st_ref[...] = jnp.zeros_like(st_ref)

    @pl.when(h == 0)
    def _():
        o_ref[...] = jnp.zeros_like(o_ref)

    acc = o_ref[0]
    for k, t in enumerate((t0, t1, t2, t3, t4)):
        acc = acc + jnp.dot(t[0], w_ref[k, 0],
                            preferred_element_type=jnp.float32)
    o_ref[0] = acc

    @pl.when(h == 1)
    def _():
        st_ref[0:1, :] += jnp.sum(acc, axis=0, keepdims=True)
        st_ref[1:2, :] += jnp.sum(acc * acc, axis=0, keepdims=True)


def _tc_matmul(txs, w2):
    tspec = pl.BlockSpec((1, BN_ROWS, 128), lambda b, n, h: (b * 2 + h, n, 0))
    return pl.pallas_call(
        _matmul_body,
        grid=(B, NT, 2),
        in_specs=[tspec] * K + [
            pl.BlockSpec((K, 1, 128, C), lambda b, n, h: (0, h, 0, 0))],
        out_specs=[
            pl.BlockSpec((1, BN_ROWS, C), lambda b, n, h: (b, n, 0)),
            pl.BlockSpec((2, C), lambda b, n, h: (0, 0)),
        ],
        out_shape=[
            jax.ShapeDtypeStruct((B, N, C), jnp.float32),
            jax.ShapeDtypeStruct((2, C), jnp.float32),
        ],
    )(*txs, w2)


def _epilogue_body(o_ref, st_ref, p_ref):
    m = 1.0 / (B * N)
    mean = st_ref[0:1, :] * m
    var = st_ref[1:2, :] * m - mean * mean
    inv = lax.rsqrt(var + EPS)
    y = (o_ref[0] - mean) * inv
    y = jnp.maximum(y, 0.0)
    y = y.reshape(N // POOL, POOL, C)
    p_ref[0] = jnp.mean(y, axis=1)


def _tc_epilogue(out, stats):
    return pl.pallas_call(
        _epilogue_body,
        grid=(B,),
        in_specs=[
            pl.BlockSpec((1, N, C), lambda b: (b, 0, 0)),
            pl.BlockSpec((2, C), lambda b: (0, 0)),
        ],
        out_specs=pl.BlockSpec((1, N // POOL, C), lambda b: (b, 0, 0)),
        out_shape=jax.ShapeDtypeStruct((B, N // POOL, C), jnp.float32),
    )(out, stats)


def kernel(x, edge_index, edge_weight, W, b):
    pad = PAD_E - E
    src = jnp.concatenate([edge_index[0].astype(jnp.int32),
                           jnp.zeros((pad,), jnp.int32)])
    dst = jnp.concatenate([edge_index[1].astype(jnp.int32),
                           jnp.zeros((pad,), jnp.int32)])
    wp = jnp.concatenate([edge_weight.astype(jnp.float32),
                          jnp.zeros((pad,), jnp.float32)])

    degrows = _sc_dega(src, dst, wp)
    dinv = _tc_dinv(degrows).reshape(NP)
    off = _sc_off(src, dst, wp, dinv)

    xq = _tc_transpose(x)
    tx = [xq]
    offb = lax.bitcast_convert_type(off, jnp.int32)
    nb = PAD_E // PEB
    epk = jnp.concatenate(
        [src.reshape(nb, PEB), dst.reshape(nb, PEB), offb.reshape(nb, PEB),
         jnp.zeros((nb, EPKW - 3 * PEB), jnp.int32)], axis=1)
    s1 = _sc_propagate(xq.reshape(QN, 128), epk)
    tx.append(s1.reshape(Q, N, 128))
    for _k in range(2, K):
        sk = _sc_propagate(tx[-1].reshape(QN, 128), epk)
        tx.append(_tc_combine(sk.reshape(Q, N, 128), tx[-2]))

    w2 = W.reshape(K, 2, 128, C)
    out, stats = _tc_matmul(tx, w2)
    return _tc_epilogue(out, stats)


# speedup vs baseline: 1.0366x; 1.0366x over previous
import jax
import jax.numpy as jnp
from jax import lax
from jax.experimental import pallas as pl
from jax.experimental.pallas import tpu as pltpu
from jax.experimental.pallas import tpu_sc as plsc

B, N, E = 4, 10000, 160000
C = 256
K = 5
EPS = 1e-5
POOL = 4
LAMBDA_MAX = 2.0

NC, NS = 2, 16
Q = B * 2
QN = Q * N

EB = 128
PAD_E = 163840
TPB_A = PAD_E // NS // EB
TPB_B = PAD_E // (NC * NS) // EB
NPT = N // NS
NPC = 125

_SCALE = 2.0 / LAMBDA_MAX

_MESH = plsc.VectorSubcoreMesh(
    core_axis_name="c", subcore_axis_name="s", num_cores=NC, num_subcores=NS)


def _iota16():
    return lax.iota(jnp.int32, 16)


def _rsqrt_sc(x):
    xi = plsc.bitcast(x, jnp.int32)
    y = plsc.bitcast(jnp.int32(0x5F3759DF) - (xi >> 1), jnp.float32)
    for _ in range(3):
        y = y * (1.5 - 0.5 * x * y * y)
    return y


NP = 10240
MY = NP // NS
EBIG = 1024
OFF_CH = PAD_E // (NC * NS) // EBIG


def _dega_body(src_hbm, dst_hbm, w_hbm, deg_hbm,
               sh_deg, bsrc, bdst, bw, bwm, rows):
    c = lax.axis_index("c")
    s = lax.axis_index("s")

    def zr(i, _):
        for u in range(8):
            rows[i, pl.ds(u * 16, 16)] = jnp.zeros((16,), jnp.float32)
        return _
    lax.fori_loop(0, EB, zr, 0)
    for p in range(MY // EB):
        pltpu.sync_copy(
            rows, sh_deg.at[pl.ds(pl.multiple_of(s * MY + p * EB, 8), EB)])
    plsc.subcore_barrier()

    def deg_batch(i, _):
        base = pl.multiple_of((c * NS + s) * (TPB_B * EB) + i * EB, 8)
        pltpu.sync_copy(src_hbm.at[pl.ds(base, EB)], bsrc)
        pltpu.sync_copy(dst_hbm.at[pl.ds(base, EB)], bdst)
        pltpu.sync_copy(w_hbm.at[pl.ds(base, EB)], bw)
        for g in range(EB // 16):
            sl = pl.ds(g * 16, 16)
            bwm[sl] = jnp.where(bsrc[sl] == bdst[sl], 0.0, bw[sl])

        def fill(e16, _):
            for t in range(16):
                e = e16 * 16 + t
                wv = plsc.load_gather(bwm, [jnp.broadcast_to(e, (16,))])
                for u in range(8):
                    rows[e, pl.ds(u * 16, 16)] = wv
            return _
        lax.fori_loop(0, EB // 16, fill, 0)
        pltpu.sync_copy(rows, sh_deg.at[bsrc], add=True)
        return _
    lax.fori_loop(0, TPB_B, deg_batch, 0)
    plsc.subcore_barrier()

    my = pl.multiple_of(s * MY, 8)
    pltpu.sync_copy(sh_deg.at[pl.ds(my, MY)],
                    deg_hbm.at[pl.ds(pl.multiple_of(c * NP + my, 8), MY)])


def _sc_dega(src, dst, w):
    return pl.kernel(
        _dega_body,
        out_type=jax.ShapeDtypeStruct((NC * NP, 128), jnp.float32),
        mesh=_MESH,
        compiler_params=pltpu.CompilerParams(needs_layout_passes=False),
        scratch_types=[
            pltpu.VMEM_SHARED((NP, 128), jnp.float32),
            pltpu.VMEM((EB,), jnp.int32),
            pltpu.VMEM((EB,), jnp.int32),
            pltpu.VMEM((EB,), jnp.float32),
            pltpu.VMEM((EB,), jnp.float32),
            pltpu.VMEM((EB, 128), jnp.float32),
        ],
    )(src, dst, w)


def _dinv_body(d_ref, o_ref):
    d = d_ref[0:NP, 0:1] + d_ref[NP:2 * NP, 0:1]
    r = lax.rsqrt(d)
    o_ref[...] = jnp.where(d > 0.0, r, 0.0)


def _tc_dinv(degrows):
    return pl.pallas_call(
        _dinv_body,
        in_specs=[pl.BlockSpec((NC * NP, 128), lambda: (0, 0))],
        out_specs=pl.BlockSpec((NP, 1), lambda: (0, 0)),
        out_shape=jax.ShapeDtypeStruct((NP, 1), jnp.float32),
    )(degrows)


def _off_body(src_hbm, dst_hbm, w_hbm, dinv_hbm, off_hbm,
              bsrc, bdst, bw, dinv_loc, boff):
    c = lax.axis_index("c")
    s = lax.axis_index("s")
    pltpu.sync_copy(dinv_hbm, dinv_loc)
    wid = c * NS + s
    def off_batch(i, _):
        ebase = pl.multiple_of(wid * (OFF_CH * EBIG) + i * EBIG, 8)
        pltpu.sync_copy(src_hbm.at[pl.ds(ebase, EBIG)], bsrc)
        pltpu.sync_copy(dst_hbm.at[pl.ds(ebase, EBIG)], bdst)
        pltpu.sync_copy(w_hbm.at[pl.ds(ebase, EBIG)], bw)
        def grp(g, _):
            sl = pl.ds(g * 16, 16)
            sv = bsrc[sl]
            dv = bdst[sl]
            wm = jnp.where(sv == dv, 0.0, bw[sl])
            g1 = plsc.load_gather(dinv_loc, [sv])
            g2 = plsc.load_gather(dinv_loc, [dv])
            boff[sl] = (-_SCALE) * g1 * wm * g2
            return _
        lax.fori_loop(0, EBIG // 16, grp, 0)
        pltpu.sync_copy(boff, off_hbm.at[pl.ds(ebase, EBIG)])
        return _
    lax.fori_loop(0, OFF_CH, off_batch, 0)


def _sc_off(src, dst, w, dinv):
    return pl.kernel(
        _off_body,
        out_type=jax.ShapeDtypeStruct((PAD_E,), jnp.float32),
        mesh=_MESH,
        compiler_params=pltpu.CompilerParams(needs_layout_passes=False),
        scratch_types=[
            pltpu.VMEM((EBIG,), jnp.int32),
            pltpu.VMEM((EBIG,), jnp.int32),
            pltpu.VMEM((EBIG,), jnp.float32),
            pltpu.VMEM((NP,), jnp.float32),
            pltpu.VMEM((EBIG,), jnp.float32),
        ],
    )(src, dst, w, dinv)


NSLOT = 4
EPT = PAD_E // NS
PEB = 64
EPKW = 256
NBT = EPT // PEB


def _prop_body(y_hbm, epk_hbm, s_hbm,
               sh_acc, gidx4, bdst4, bpk4, rows4,
               gs0, gs1, gs2, gs3, ss0, ss1, ss2, ss3,
               ps0, ps1, ps2, ps3):
    c = lax.axis_index("c")
    s = lax.axis_index("s")
    gsems = (gs0, gs1, gs2, gs3)
    ssems = (ss0, ss1, ss2, ss3)
    psems = (ps0, ps1, ps2, ps3)

    my0 = pl.multiple_of(s * 624, 8)
    gb0 = s * NBT

    def pstart(ib, k):
        pltpu.async_copy(epk_hbm.at[gb0 + ib], bpk4.at[k], psems[k])

    def pwait(k):
        pltpu.make_async_copy(epk_hbm.at[gb0], bpk4.at[k], psems[k]).wait()


    def gstart(k):
        pltpu.async_copy(y_hbm.at[gidx4.at[k]], rows4.at[k], gsems[k])

    def gwait(k):
        pltpu.make_async_copy(y_hbm.at[gidx4.at[k]], rows4.at[k],
                              gsems[k]).wait()

    def sstart(k):
        pltpu.async_copy(rows4.at[k], sh_acc.at[bdst4.at[k]], ssems[k],
                         add=True)

    def swait(k):
        pltpu.make_async_copy(rows4.at[k], sh_acc.at[bdst4.at[k]],
                              ssems[k]).wait()

    def scale(k):
        k16 = jnp.full((16,), k, jnp.int32)

        def scale_e16(e16, _):
            for t in range(16):
                e = e16 * 16 + t
                ov = plsc.bitcast(
                    plsc.load_gather(
                        bpk4, [k16, jnp.broadcast_to(2 * PEB + e, (16,))]),
                    jnp.float32)
                for u in range(8):
                    cs = pl.ds(u * 16, 16)
                    rows4[k, e, cs] = rows4[k, e, cs] * ov
            return _
        lax.fori_loop(0, PEB // 16, scale_e16, 0)

    def chunk(j, carry):
        q = c * (Q // NC) + j
        qb = pl.multiple_of(q * N, 8)
        qn = q * N

        def zz(r, _):
            for u in range(8):
                rows4[0, r, pl.ds(u * 16, 16)] = jnp.zeros((16,), jnp.float32)
            return _
        lax.fori_loop(0, PEB, zz, 0)

        @pl.when(s < NS - 1)
        def _():
            for p in range(9):
                pltpu.async_copy(rows4.at[0],
                                 sh_acc.at[pl.ds(my0 + p * PEB, PEB)], gs0)
            pltpu.async_copy(rows4.at[0, pl.ds(0, 48)],
                             sh_acc.at[pl.ds(my0 + 576, 48)], gs0)
            for p in range(9):
                pltpu.make_async_copy(
                    rows4.at[0],
                    sh_acc.at[pl.ds(my0 + p * PEB, PEB)], gs0).wait()
            pltpu.make_async_copy(rows4.at[0, pl.ds(0, 48)],
                                  sh_acc.at[pl.ds(my0 + 576, 48)], gs0).wait()

        @pl.when(s == NS - 1)
        def _():
            for p in range(10):
                pltpu.async_copy(rows4.at[0],
                                 sh_acc.at[pl.ds(9360 + p * PEB, PEB)], gs0)
            for p in range(10):
                pltpu.make_async_copy(
                    rows4.at[0],
                    sh_acc.at[pl.ds(9360 + p * PEB, PEB)], gs0).wait()
        plsc.subcore_barrier()

        def buildq(ib, k):
            for g in range(PEB // 16):
                sl = pl.ds(g * 16, 16)
                gidx4[k, sl] = bpk4[k, pl.ds(g * 16, 16)] + qn
                bdst4[k, sl] = bpk4[k, pl.ds(PEB + g * 16, 16)]

        for k in range(NSLOT):
            pstart(jnp.int32(k), k)
        for k in range(2):
            pwait(k)
            buildq(jnp.int32(k), k)
            gstart(k)

        def quad(i4, carry2):
            for u in range(NSLOT):
                i = i4 * NSLOT + u
                kn = (u + 2) % NSLOT

                @pl.when(i >= 2)
                def _():
                    swait(kn)

                @pl.when(i + 2 < NBT)
                def _():
                    pwait(kn)
                    buildq(i + 2, kn)
                    gstart(kn)
                gwait(u)
                scale(u)
                sstart(u)

                @pl.when(i + NSLOT < NBT)
                def _():
                    pstart(i + NSLOT, u)
            return carry2
        lax.fori_loop(0, NBT // NSLOT, quad, 0)
        for k in (2, 3):
            swait(k)
        plsc.subcore_barrier()

        @pl.when(s < NS - 1)
        def _():
            pltpu.sync_copy(sh_acc.at[pl.ds(my0, 624)],
                            s_hbm.at[pl.ds(pl.multiple_of(qb + my0, 8), 624)])

        @pl.when(s == NS - 1)
        def _():
            pltpu.sync_copy(sh_acc.at[pl.ds(9360, 640)],
                            s_hbm.at[pl.ds(pl.multiple_of(qb + 9360, 8), 640)])
        plsc.subcore_barrier()
        return carry
    lax.fori_loop(0, Q // NC, chunk, 0)


def _sc_propagate(y2d, epk):
    return pl.kernel(
        _prop_body,
        out_type=jax.ShapeDtypeStruct((QN, 128), jnp.float32),
        mesh=_MESH,
        compiler_params=pltpu.CompilerParams(needs_layout_passes=False),
        scratch_types=[
            pltpu.VMEM_SHARED((N, 128), jnp.float32),
            pltpu.VMEM((NSLOT, PEB), jnp.int32),
            pltpu.VMEM((NSLOT, PEB), jnp.int32),
            pltpu.VMEM((NSLOT, EPKW), jnp.int32),
            pltpu.VMEM((NSLOT, PEB, 128), jnp.float32),
            pltpu.SemaphoreType.DMA, pltpu.SemaphoreType.DMA,
            pltpu.SemaphoreType.DMA, pltpu.SemaphoreType.DMA,
            pltpu.SemaphoreType.DMA, pltpu.SemaphoreType.DMA,
            pltpu.SemaphoreType.DMA, pltpu.SemaphoreType.DMA,
            pltpu.SemaphoreType.DMA, pltpu.SemaphoreType.DMA,
            pltpu.SemaphoreType.DMA, pltpu.SemaphoreType.DMA,
        ],
    )(y2d, epk)


BN_ROWS = 2000
NT = N // BN_ROWS


def _transpose_body(x_ref, o_ref):
    o_ref[...] = x_ref[...]


def _tc_transpose(x):
    return pl.pallas_call(
        _transpose_body,
        grid=(Q, NT),
        in_specs=[pl.BlockSpec((1, BN_ROWS, 128),
                               lambda q, n: (q // 2, n, q % 2))],
        out_specs=pl.BlockSpec((1, BN_ROWS, 128), lambda q, n: (q, n, 0)),
        out_shape=jax.ShapeDtypeStruct((Q, N, 128), jnp.float32),
    )(x)


def _combine_body(s_ref, y0_ref, o_ref):
    o_ref[...] = 2.0 * s_ref[...] - y0_ref[...]


def _tc_combine(s_new, y0):
    spec = pl.BlockSpec((1, BN_ROWS, 128), lambda q, n: (q, n, 0))
    return pl.pallas_call(
        _combine_body,
        grid=(Q, NT),
        in_specs=[spec, spec],
        out_specs=spec,
        out_shape=jax.ShapeDtypeStruct((Q, N, 128), jnp.float32),
    )(s_new, y0)


def _matmul_body(t0, t1, t2, t3, t4, w_ref, o_ref, st_ref):
    b = pl.program_id(0)
    n = pl.program_id(1)
    h = pl.program_id(2)

    @pl.when(jnp.logical_and(jnp.logical_and(b == 0, n == 0), h == 0))
    def _():
        st_ref[...] = jnp.zeros_like(st_ref)

    @pl.when(h == 0)
    def _():
        o_ref[...] = jnp.zeros_like(o_ref)

    acc = o_ref[0]
    for k, t in enumerate((t0, t1, t2, t3, t4)):
        acc = acc + jnp.dot(t[0], w_ref[k, 0],
                            preferred_element_type=jnp.float32)
    o_ref[0] = acc

    @pl.when(h == 1)
    def _():
        st_ref[0:1, :] += jnp.sum(acc, axis=0, keepdims=True)
        st_ref[1:2, :] += jnp.sum(acc * acc, axis=0, keepdims=True)


def _tc_matmul(txs, w2):
    tspec = pl.BlockSpec((1, BN_ROWS, 128), lambda b, n, h: (b * 2 + h, n, 0))
    return pl.pallas_call(
        _matmul_body,
        grid=(B, NT, 2),
        in_specs=[tspec] * K + [
            pl.BlockSpec((K, 1, 128, C), lambda b, n, h: (0, h, 0, 0))],
        out_specs=[
            pl.BlockSpec((1, BN_ROWS, C), lambda b, n, h: (b, n, 0)),
            pl.BlockSpec((2, C), lambda b, n, h: (0, 0)),
        ],
        out_shape=[
            jax.ShapeDtypeStruct((B, N, C), jnp.float32),
            jax.ShapeDtypeStruct((2, C), jnp.float32),
        ],
    )(*txs, w2)


def _epilogue_body(o_ref, st_ref, p_ref):
    m = 1.0 / (B * N)
    mean = st_ref[0:1, :] * m
    var = st_ref[1:2, :] * m - mean * mean
    inv = lax.rsqrt(var + EPS)
    y = (o_ref[0] - mean) * inv
    y = jnp.maximum(y, 0.0)
    y = y.reshape(N // POOL, POOL, C)
    p_ref[0] = jnp.mean(y, axis=1)


def _tc_epilogue(out, stats):
    return pl.pallas_call(
        _epilogue_body,
        grid=(B,),
        in_specs=[
            pl.BlockSpec((1, N, C), lambda b: (b, 0, 0)),
            pl.BlockSpec((2, C), lambda b: (0, 0)),
        ],
        out_specs=pl.BlockSpec((1, N // POOL, C), lambda b: (b, 0, 0)),
        out_shape=jax.ShapeDtypeStruct((B, N // POOL, C), jnp.float32),
    )(out, stats)


def kernel(x, edge_index, edge_weight, W, b):
    pad = PAD_E - E
    src = jnp.concatenate([edge_index[0].astype(jnp.int32),
                           jnp.zeros((pad,), jnp.int32)])
    dst = jnp.concatenate([edge_index[1].astype(jnp.int32),
                           jnp.zeros((pad,), jnp.int32)])
    wp = jnp.concatenate([edge_weight.astype(jnp.float32),
                          jnp.zeros((pad,), jnp.float32)])

    degrows = _sc_dega(src, dst, wp)
    dinv = _tc_dinv(degrows).reshape(NP)
    off = _sc_off(src, dst, wp, dinv)

    xq = _tc_transpose(x)
    tx = [xq]
    offb = lax.bitcast_convert_type(off, jnp.int32)
    nb = PAD_E // PEB
    epk = jnp.concatenate(
        [src.reshape(nb, PEB), dst.reshape(nb, PEB), offb.reshape(nb, PEB),
         jnp.zeros((nb, EPKW - 3 * PEB), jnp.int32)], axis=1)
    s1 = _sc_propagate(xq.reshape(QN, 128), epk)
    tx.append(s1.reshape(Q, N, 128))
    for _k in range(2, K):
        sk = _sc_propagate(tx[-1].reshape(QN, 128), epk)
        tx.append(_tc_combine(sk.reshape(Q, N, 128), tx[-2]))

    w2 = W.reshape(K, 2, 128, C)
    out, stats = _tc_matmul(tx, w2)
    return _tc_epilogue(out, stats)
